# baseline (speedup 1.0000x reference)
import functools

import jax
import jax.numpy as jnp
from jax import lax
from jax.experimental import pallas as pl
from jax.experimental.pallas import tpu as pltpu
from jax.experimental.pallas import tpu_sc as plsc

_NC = 2
_NS = 16
_NW = _NC * _NS

_B, _T, _D = 4, 8192, 1024
_K = _T // 2
_ROWS = _B * _K
_RPW = _ROWS // _NW
_CH = 32
_NCHUNK = _RPW // _CH


_NV = _T // 16
_RADIX_BINS = 256 * 16


def _sort_body(scores_hbm, idx_hbm, sc_v, k0, v0, k1, v1, bins, outb):
    wid = lax.axis_index("s") * _NC + lax.axis_index("c")

    @pl.when(wid < _B)
    def _():
        r = wid
        pltpu.sync_copy(scores_hbm.at[r], sc_v)
        lane = lax.iota(jnp.int32, 16)
        ones = jnp.ones((16,), jnp.int32)
        zeros = jnp.zeros((16,), jnp.int32)

        def build(i, c):
            f = sc_v[pl.ds(i * 16, 16)]
            u = lax.bitcast_convert_type(f, jnp.int32)
            m = lax.shift_right_arithmetic(u, 31)
            key = u ^ (jnp.bitwise_not(m) & jnp.int32(0x7FFFFFFF))
            p = i * 16 + lane
            a = (p & (_NV - 1)) * 16 + lax.shift_right_logical(p, 9)
            plsc.store_scatter(k0, [a], key)
            plsc.store_scatter(v0, [a], p)
            return c

        lax.fori_loop(0, _NV, build, 0)

        bufs = ((k0, v0, k1, v1), (k1, v1, k0, v0))
        for pas in range(4):
            kin, vin, kout, vout = bufs[pas % 2]
            sh = 8 * pas

            def zb(i, c):
                bins[pl.ds(i * 16, 16)] = zeros
                return c

            lax.fori_loop(0, _RADIX_BINS // 16, zb, 0)

            def hist(i, c):
                k = kin[pl.ds(i * 16, 16)]
                d = (lax.shift_right_logical(k, sh) & jnp.int32(0xFF)) * 16 + lane
                plsc.addupdate_scatter(bins, [d], ones)
                return c

            lax.fori_loop(0, _NV, hist, 0)

            def scan(i, run):
                v = bins[pl.ds(i * 16, 16)]
                cs = plsc.cumsum(v)
                bins[pl.ds(i * 16, 16)] = (cs - v) + run
                return run + jnp.sum(v)

            lax.fori_loop(0, _RADIX_BINS // 16, scan, jnp.int32(0))

            if pas < 3:

                def perm(i, c):
                    k = kin[pl.ds(i * 16, 16)]
                    val = vin[pl.ds(i * 16, 16)]
                    d = (lax.shift_right_logical(k, sh) & jnp.int32(0xFF)) * 16 + lane
                    q = plsc.load_gather(bins, [d])
                    a = (q & (_NV - 1)) * 16 + lax.shift_right_logical(q, 9)
                    plsc.store_scatter(kout, [a], k)
                    plsc.store_scatter(vout, [a], val)
                    plsc.store_scatter(bins, [d], q + 1)
                    return c

            else:
                def perm(i, c):
                    k = kin[pl.ds(i * 16, 16)]
                    val = vin[pl.ds(i * 16, 16)]
                    d = (lax.shift_right_logical(k, sh) & jnp.int32(0xFF)) * 16 + lane
                    q = plsc.load_gather(bins, [d])
                    plsc.store_scatter(outb, [q], val, mask=q < _K)
                    plsc.store_scatter(bins, [d], q + 1)
                    return c

            lax.fori_loop(0, _NV, perm, 0)

        pltpu.sync_copy(outb, idx_hbm.at[r])


def _sc_sort(rand_scores):
    mesh = plsc.VectorSubcoreMesh(
        core_axis_name="c", subcore_axis_name="s", num_cores=_NC, num_subcores=_NS
    )
    return pl.kernel(
        _sort_body,
        out_type=jax.ShapeDtypeStruct((_B, _K), jnp.int32),
        mesh=mesh,
        compiler_params=pltpu.CompilerParams(
            use_tc_tiling_on_sc=False, needs_layout_passes=False
        ),
        scratch_types=[
            pltpu.VMEM((_T,), jnp.float32),
            pltpu.VMEM((_T,), jnp.int32),
            pltpu.VMEM((_T,), jnp.int32),
            pltpu.VMEM((_T,), jnp.int32),
            pltpu.VMEM((_T,), jnp.int32),
            pltpu.VMEM((_RADIX_BINS,), jnp.int32),
            pltpu.VMEM((_K,), jnp.int32),
        ],
    )(rand_scores)


def _gather_body(x_hbm, idx_hbm, out_hbm, idx_v, buf0, buf1, sem0, sem1):
    wid = lax.axis_index("s") * _NC + lax.axis_index("c")
    base = wid * _RPW
    pltpu.sync_copy(idx_hbm.at[wid], idx_v)

    bufs = (buf0, buf1)
    sems = (sem0, sem1)
    d0 = pltpu.async_copy(x_hbm.at[idx_v.at[0]], bufs[0], sems[0])
    descs = [d0, None]
    for c in range(_NCHUNK):
        descs[c % 2].wait()
        if c + 1 < _NCHUNK:
            descs[(c + 1) % 2] = pltpu.async_copy(
                x_hbm.at[idx_v.at[c + 1]], bufs[(c + 1) % 2], sems[(c + 1) % 2]
            )
        pltpu.sync_copy(bufs[c % 2], out_hbm.at[pl.ds(base + c * _CH, _CH)])


def _sc_gather(x_flat, idx_chunked):
    mesh = plsc.VectorSubcoreMesh(
        core_axis_name="c", subcore_axis_name="s", num_cores=_NC, num_subcores=_NS
    )
    return pl.kernel(
        _gather_body,
        out_type=jax.ShapeDtypeStruct((_ROWS, _D), jnp.float32),
        mesh=mesh,
        scratch_types=[
            pltpu.VMEM((_NCHUNK, _CH), jnp.int32),
            pltpu.VMEM((_CH, _D), jnp.float32),
            pltpu.VMEM((_CH, _D), jnp.float32),
            pltpu.SemaphoreType.DMA,
            pltpu.SemaphoreType.DMA,
        ],
    )(x_flat, idx_chunked)


def kernel(x, rand_scores):
    B, T, D = x.shape
    num_keep = _K
    token_indices_keep = _sc_sort(rand_scores)
    gidx = token_indices_keep + (jnp.arange(B, dtype=jnp.int32) * T)[:, None]
    gidx = gidx.reshape(_NW, _NCHUNK, _CH)
    out = _sc_gather(x.reshape(B * T, D), gidx)
    return (out.reshape(B, num_keep, D), token_indices_keep)


# speedup vs baseline: 1.0384x; 1.0384x over previous
import functools

import jax
import jax.numpy as jnp
from jax import lax
from jax.experimental import pallas as pl
from jax.experimental.pallas import tpu as pltpu
from jax.experimental.pallas import tpu_sc as plsc

_NC = 2
_NS = 16
_NW = _NC * _NS

_B, _T, _D = 4, 8192, 1024
_K = _T // 2
_ROWS = _B * _K
_RPW = _ROWS // _NW
_CH = 32
_NCHUNK = _RPW // _CH


_NV = _T // 16
_RADIX_BINS = 256 * 16


def _sort_body(scores_hbm, idx_hbm, sc_v, k0, v0, k1, v1, bins, outb):
    wid = lax.axis_index("s") * _NC + lax.axis_index("c")

    @pl.when(wid < _B)
    def _():
        r = wid
        pltpu.sync_copy(scores_hbm.at[r], sc_v)
        lane = lax.iota(jnp.int32, 16)
        ones = jnp.ones((16,), jnp.int32)
        zeros = jnp.zeros((16,), jnp.int32)

        def build(i, c):
            f = sc_v[pl.ds(i * 16, 16)]
            u = lax.bitcast_convert_type(f, jnp.int32)
            m = lax.shift_right_arithmetic(u, 31)
            key = u ^ (jnp.bitwise_not(m) & jnp.int32(0x7FFFFFFF))
            p = i * 16 + lane
            a = (p & (_NV - 1)) * 16 + lax.shift_right_logical(p, 9)
            plsc.store_scatter(k0, [a], key)
            plsc.store_scatter(v0, [a], p)
            return c

        lax.fori_loop(0, _NV, build, 0, unroll=4)

        bufs = ((k0, v0, k1, v1), (k1, v1, k0, v0))
        for pas in range(4):
            kin, vin, kout, vout = bufs[pas % 2]
            sh = 8 * pas

            def zb(i, c):
                bins[pl.ds(i * 16, 16)] = zeros
                return c

            lax.fori_loop(0, _RADIX_BINS // 16, zb, 0, unroll=4)

            def hist(i, c):
                k = kin[pl.ds(i * 16, 16)]
                d = (lax.shift_right_logical(k, sh) & jnp.int32(0xFF)) * 16 + lane
                plsc.addupdate_scatter(bins, [d], ones)
                return c

            lax.fori_loop(0, _NV, hist, 0, unroll=4)

            def scan(i, run):
                v = bins[pl.ds(i * 16, 16)]
                cs = plsc.cumsum(v)
                bins[pl.ds(i * 16, 16)] = (cs - v) + run
                return run + jnp.sum(v)

            lax.fori_loop(0, _RADIX_BINS // 16, scan, jnp.int32(0))

            if pas < 3:

                def perm(i, c):
                    k = kin[pl.ds(i * 16, 16)]
                    val = vin[pl.ds(i * 16, 16)]
                    d = (lax.shift_right_logical(k, sh) & jnp.int32(0xFF)) * 16 + lane
                    q = plsc.load_gather(bins, [d])
                    a = (q & (_NV - 1)) * 16 + lax.shift_right_logical(q, 9)
                    plsc.store_scatter(kout, [a], k)
                    plsc.store_scatter(vout, [a], val)
                    plsc.store_scatter(bins, [d], q + 1)
                    return c

            else:
                def perm(i, c):
                    k = kin[pl.ds(i * 16, 16)]
                    val = vin[pl.ds(i * 16, 16)]
                    d = (lax.shift_right_logical(k, sh) & jnp.int32(0xFF)) * 16 + lane
                    q = plsc.load_gather(bins, [d])
                    plsc.store_scatter(outb, [q], val, mask=q < _K)
                    plsc.store_scatter(bins, [d], q + 1)
                    return c

            lax.fori_loop(0, _NV, perm, 0, unroll=4)

        pltpu.sync_copy(outb, idx_hbm.at[r])


def _sc_sort(rand_scores):
    mesh = plsc.VectorSubcoreMesh(
        core_axis_name="c", subcore_axis_name="s", num_cores=_NC, num_subcores=_NS
    )
    return pl.kernel(
        _sort_body,
        out_type=jax.ShapeDtypeStruct((_B, _K), jnp.int32),
        mesh=mesh,
        compiler_params=pltpu.CompilerParams(
            use_tc_tiling_on_sc=False, needs_layout_passes=False
        ),
        scratch_types=[
            pltpu.VMEM((_T,), jnp.float32),
            pltpu.VMEM((_T,), jnp.int32),
            pltpu.VMEM((_T,), jnp.int32),
            pltpu.VMEM((_T,), jnp.int32),
            pltpu.VMEM((_T,), jnp.int32),
            pltpu.VMEM((_RADIX_BINS,), jnp.int32),
            pltpu.VMEM((_K,), jnp.int32),
        ],
    )(rand_scores)


def _gather_body(x_hbm, idx_hbm, out_hbm, idx_v, buf0, buf1, sem0, sem1):
    wid = lax.axis_index("s") * _NC + lax.axis_index("c")
    base = wid * _RPW
    pltpu.sync_copy(idx_hbm.at[wid], idx_v)

    bufs = (buf0, buf1)
    sems = (sem0, sem1)
    d0 = pltpu.async_copy(x_hbm.at[idx_v.at[0]], bufs[0], sems[0])
    descs = [d0, None]
    for c in range(_NCHUNK):
        descs[c % 2].wait()
        if c + 1 < _NCHUNK:
            descs[(c + 1) % 2] = pltpu.async_copy(
                x_hbm.at[idx_v.at[c + 1]], bufs[(c + 1) % 2], sems[(c + 1) % 2]
            )
        pltpu.sync_copy(bufs[c % 2], out_hbm.at[pl.ds(base + c * _CH, _CH)])


def _sc_gather(x_flat, idx_chunked):
    mesh = plsc.VectorSubcoreMesh(
        core_axis_name="c", subcore_axis_name="s", num_cores=_NC, num_subcores=_NS
    )
    return pl.kernel(
        _gather_body,
        out_type=jax.ShapeDtypeStruct((_ROWS, _D), jnp.float32),
        mesh=mesh,
        scratch_types=[
            pltpu.VMEM((_NCHUNK, _CH), jnp.int32),
            pltpu.VMEM((_CH, _D), jnp.float32),
            pltpu.VMEM((_CH, _D), jnp.float32),
            pltpu.SemaphoreType.DMA,
            pltpu.SemaphoreType.DMA,
        ],
    )(x_flat, idx_chunked)


def kernel(x, rand_scores):
    B, T, D = x.shape
    num_keep = _K
    token_indices_keep = _sc_sort(rand_scores)
    gidx = token_indices_keep + (jnp.arange(B, dtype=jnp.int32) * T)[:, None]
    gidx = gidx.reshape(_NW, _NCHUNK, _CH)
    out = _sc_gather(x.reshape(B * T, D), gidx)
    return (out.reshape(B, num_keep, D), token_indices_keep)
